# initial kernel scaffold (unmeasured)
import jax
import jax.numpy as jnp
from jax import lax
from jax.experimental import pallas as pl
from jax.experimental.pallas import tpu as pltpu

N_DEV = 4


def kernel(x, w_mat):
    m_total, k_per = x.shape
    _, n = w_mat.shape
    m_per = m_total // N_DEV

    def body(x_ref, w_ref, out_ref, sbuf, rbuf, abuf,
             send_sems, recv_sems, a_send_sems, a_recv_sems):
        my = lax.axis_index("i")
        left = lax.rem(my + N_DEV - 1, N_DEV)
        right = lax.rem(my + 1, N_DEV)

        barrier = pltpu.get_barrier_semaphore()
        for nbr in (left, right):
            pl.semaphore_signal(barrier, inc=1, device_id=(nbr,),
                                device_id_type=pl.DeviceIdType.MESH)
        pl.semaphore_wait(barrier, 2)

        w = w_ref[...].astype(jnp.bfloat16)

        def partial_chunk(c):
            xc = x_ref[pl.ds(c * m_per, m_per), :].astype(jnp.bfloat16)
            return jnp.dot(xc, w, preferred_element_type=jnp.float32)

        for s in range(N_DEV - 1):
            c = lax.rem(my + 2 * N_DEV - 1 - s, N_DEV)
            part = partial_chunk(c)
            if s == 0:
                sbuf[...] = part.astype(jnp.bfloat16)
            else:
                sbuf[...] = (rbuf[s - 1].astype(jnp.float32)
                             + part).astype(jnp.bfloat16)
            rdma = pltpu.make_async_remote_copy(
                src_ref=sbuf, dst_ref=rbuf.at[s],
                send_sem=send_sems.at[s], recv_sem=recv_sems.at[s],
                device_id=(right,), device_id_type=pl.DeviceIdType.MESH)
            rdma.start()
            rdma.wait()

        y = rbuf[N_DEV - 2].astype(jnp.float32) + partial_chunk(my)
        y = jnp.maximum(y, 0.0)

        abuf[0] = jnp.full((8, 128), jnp.max(y), jnp.float32)
        for t, partner in enumerate((my ^ 1, 3 - my)):
            rdma = pltpu.make_async_remote_copy(
                src_ref=abuf.at[0], dst_ref=abuf.at[t + 1],
                send_sem=a_send_sems.at[t], recv_sem=a_recv_sems.at[t],
                device_id=(partner,), device_id_type=pl.DeviceIdType.MESH)
            rdma.start()
            rdma.wait()
            abuf[0] = jnp.full(
                (8, 128),
                jnp.maximum(jnp.max(abuf[0]), jnp.max(abuf[t + 1])),
                jnp.float32)

        scale = jnp.max(abuf[0]) / 448.0
        q = (y / scale).astype(jnp.float8_e4m3fn)
        out_ref[...] = q.astype(jnp.float32) * scale

    return pl.pallas_call(
        body,
        out_shape=jax.ShapeDtypeStruct((m_per, n), jnp.float32),
        in_specs=[pl.BlockSpec(memory_space=pltpu.VMEM),
                  pl.BlockSpec(memory_space=pltpu.VMEM)],
        out_specs=pl.BlockSpec(memory_space=pltpu.VMEM),
        scratch_shapes=[
            pltpu.VMEM((m_per, n), jnp.bfloat16),
            pltpu.VMEM((N_DEV - 1, m_per, n), jnp.bfloat16),
            pltpu.VMEM((3, 8, 128), jnp.float32),
            pltpu.SemaphoreType.DMA((N_DEV - 1,)),
            pltpu.SemaphoreType.DMA((N_DEV - 1,)),
            pltpu.SemaphoreType.DMA((2,)),
            pltpu.SemaphoreType.DMA((2,)),
        ],
        compiler_params=pltpu.CompilerParams(collective_id=0),
    )(x, w_mat)


# baseline (device time: 187393 ns/iter reference)
import jax
import jax.numpy as jnp
from jax import lax
from jax.experimental import pallas as pl
from jax.experimental.pallas import tpu as pltpu

N_DEV = 4


def kernel(x, w_mat):
    m_total, k_per = x.shape
    _, n = w_mat.shape
    m_per = m_total // N_DEV

    def body(x_ref, w_ref, out_ref, sbuf, rbuf, abuf,
             send_sems, recv_sems, a_send_sems, a_recv_sems):
        my = lax.axis_index("i")
        left = lax.rem(my + N_DEV - 1, N_DEV)
        right = lax.rem(my + 1, N_DEV)

        barrier = pltpu.get_barrier_semaphore()
        for nbr in (left, right):
            pl.semaphore_signal(barrier, inc=1, device_id=(nbr,),
                                device_id_type=pl.DeviceIdType.MESH)
        pl.semaphore_wait(barrier, 2)

        w = w_ref[...].astype(jnp.bfloat16)

        def partial_chunk(c):
            xc = x_ref[pl.ds(c * m_per, m_per), :].astype(jnp.bfloat16)
            return jnp.dot(xc, w, preferred_element_type=jnp.float32)

        for s in range(N_DEV - 1):
            c = lax.rem(my + 2 * N_DEV - 1 - s, N_DEV)
            part = partial_chunk(c)
            if s == 0:
                sbuf[...] = part.astype(jnp.bfloat16)
            else:
                sbuf[...] = (rbuf[s - 1].astype(jnp.float32)
                             + part).astype(jnp.bfloat16)
            rdma = pltpu.make_async_remote_copy(
                src_ref=sbuf, dst_ref=rbuf.at[s],
                send_sem=send_sems.at[s], recv_sem=recv_sems.at[s],
                device_id=(right,), device_id_type=pl.DeviceIdType.MESH)
            rdma.start()
            rdma.wait()

        y = rbuf[N_DEV - 2].astype(jnp.float32) + partial_chunk(my)
        y = jnp.maximum(y, 0.0)

        abuf[0] = jnp.full((8, 128), jnp.max(y), jnp.float32)
        for t, partner in enumerate((my ^ 1, 3 - my)):
            rdma = pltpu.make_async_remote_copy(
                src_ref=abuf.at[0], dst_ref=abuf.at[t + 1],
                send_sem=a_send_sems.at[t], recv_sem=a_recv_sems.at[t],
                device_id=(partner,), device_id_type=pl.DeviceIdType.MESH)
            rdma.start()
            rdma.wait()
            abuf[0] = jnp.full(
                (8, 128),
                jnp.maximum(jnp.max(abuf[0]), jnp.max(abuf[t + 1])),
                jnp.float32)

        scale = jnp.max(abuf[0]) / 448.0
        q = (y / scale).astype(jnp.float8_e4m3fn)
        out_ref[...] = q.astype(jnp.float32) * scale

    return pl.pallas_call(
        body,
        out_shape=jax.ShapeDtypeStruct((m_per, n), jnp.float32),
        in_specs=[pl.BlockSpec(memory_space=pltpu.VMEM),
                  pl.BlockSpec(memory_space=pltpu.VMEM)],
        out_specs=pl.BlockSpec(memory_space=pltpu.VMEM),
        scratch_shapes=[
            pltpu.VMEM((m_per, n), jnp.bfloat16),
            pltpu.VMEM((N_DEV - 1, m_per, n), jnp.bfloat16),
            pltpu.VMEM((3, 8, 128), jnp.float32),
            pltpu.SemaphoreType.DMA((N_DEV - 1,)),
            pltpu.SemaphoreType.DMA((N_DEV - 1,)),
            pltpu.SemaphoreType.DMA((2,)),
            pltpu.SemaphoreType.DMA((2,)),
        ],
        compiler_params=pltpu.CompilerParams(
            collective_id=0, vmem_limit_bytes=100 * 1024 * 1024),
    )(x, w_mat)


# device time: 109186 ns/iter; 1.7163x vs baseline; 1.7163x over previous
import jax
import jax.numpy as jnp
from jax import lax
from jax.experimental import pallas as pl
from jax.experimental.pallas import tpu as pltpu

N_DEV = 4


def kernel(x, w_mat):
    m_total, k_per = x.shape
    _, n = w_mat.shape
    m_per = m_total // N_DEV
    h = n // 2

    def body(x_ref, w_ref, out_ref, sbuf_cw, sbuf_ccw, rbuf_cw, rbuf_ccw,
             abuf, cw_send, cw_recv, ccw_send, ccw_recv, a_send, a_recv):
        my = lax.axis_index("i")
        left = lax.rem(my + N_DEV - 1, N_DEV)
        right = lax.rem(my + 1, N_DEV)

        barrier = pltpu.get_barrier_semaphore()
        for nbr in (left, right):
            pl.semaphore_signal(barrier, inc=1, device_id=(nbr,),
                                device_id_type=pl.DeviceIdType.MESH)
        pl.semaphore_wait(barrier, 2)

        def part(c, lo):
            xc = x_ref[pl.ds(c * m_per, m_per), :].astype(jnp.bfloat16)
            wh = (w_ref[:, :h] if lo else w_ref[:, h:]).astype(jnp.bfloat16)
            return jnp.dot(
                xc, wh, preferred_element_type=jnp.float32
            ).astype(jnp.bfloat16)

        def acc(rbuf_slot, p):
            return (rbuf_slot.astype(jnp.float32)
                    + p.astype(jnp.float32)).astype(jnp.bfloat16)

        sbuf_cw[0] = part(lax.rem(my + N_DEV - 1, N_DEV), True)
        sbuf_ccw[0] = part(lax.rem(my + 1, N_DEV), False)

        descs = []
        amax = None
        for s in range(N_DEV - 1):
            cw = pltpu.make_async_remote_copy(
                src_ref=sbuf_cw.at[s % 2], dst_ref=rbuf_cw.at[s],
                send_sem=cw_send.at[s], recv_sem=cw_recv.at[s],
                device_id=(right,), device_id_type=pl.DeviceIdType.MESH)
            ccw = pltpu.make_async_remote_copy(
                src_ref=sbuf_ccw.at[s % 2], dst_ref=rbuf_ccw.at[s],
                send_sem=ccw_send.at[s], recv_sem=ccw_recv.at[s],
                device_id=(left,), device_id_type=pl.DeviceIdType.MESH)
            cw.start()
            ccw.start()
            descs.append((cw, ccw))
            if s < N_DEV - 2:
                c_lo = lax.rem(my + 2 * N_DEV - 2 - s, N_DEV)
                c_hi = lax.rem(my + 2 + s, N_DEV)
            else:
                c_lo = c_hi = my
            p_lo = part(c_lo, True)
            p_hi = part(c_hi, False)
            if s < N_DEV - 2:
                if s >= 1:
                    descs[s - 1][0].wait_send()
                    descs[s - 1][1].wait_send()
                cw.wait_recv()
                sbuf_cw[(s + 1) % 2] = acc(rbuf_cw[s], p_lo)
                ccw.wait_recv()
                sbuf_ccw[(s + 1) % 2] = acc(rbuf_ccw[s], p_hi)
            else:
                cw.wait_recv()
                y_lo = jnp.maximum(rbuf_cw[s].astype(jnp.float32)
                                   + p_lo.astype(jnp.float32), 0.0)
                amax = jnp.max(y_lo)
                out_ref[:, :h] = y_lo
                ccw.wait_recv()
                y_hi = jnp.maximum(rbuf_ccw[s].astype(jnp.float32)
                                   + p_hi.astype(jnp.float32), 0.0)
                amax = jnp.maximum(amax, jnp.max(y_hi))
                out_ref[:, h:] = y_hi
        for cw_d, ccw_d in descs[1:]:
            cw_d.wait_send()
            ccw_d.wait_send()

        abuf[0] = jnp.full((8, 128), amax, jnp.float32)
        for t, partner in enumerate((my ^ 1, 3 - my)):
            rdma = pltpu.make_async_remote_copy(
                src_ref=abuf.at[0], dst_ref=abuf.at[t + 1],
                send_sem=a_send.at[t], recv_sem=a_recv.at[t],
                device_id=(partner,), device_id_type=pl.DeviceIdType.MESH)
            rdma.start()
            rdma.wait()
            abuf[0] = jnp.full(
                (8, 128),
                jnp.maximum(jnp.max(abuf[0]), jnp.max(abuf[t + 1])),
                jnp.float32)

        scale = jnp.max(abuf[0]) / 448.0
        inv = 1.0 / scale
        q = n // 4
        for i in range(4):
            blk = out_ref[:, i * q:(i + 1) * q]
            qv = (blk * inv).astype(jnp.float8_e4m3fn)
            out_ref[:, i * q:(i + 1) * q] = qv.astype(jnp.float32) * scale

    return pl.pallas_call(
        body,
        out_shape=jax.ShapeDtypeStruct((m_per, n), jnp.float32),
        in_specs=[pl.BlockSpec(memory_space=pltpu.VMEM),
                  pl.BlockSpec(memory_space=pltpu.VMEM)],
        out_specs=pl.BlockSpec(memory_space=pltpu.VMEM),
        scratch_shapes=[
            pltpu.VMEM((2, m_per, h), jnp.bfloat16),
            pltpu.VMEM((2, m_per, h), jnp.bfloat16),
            pltpu.VMEM((N_DEV - 1, m_per, h), jnp.bfloat16),
            pltpu.VMEM((N_DEV - 1, m_per, h), jnp.bfloat16),
            pltpu.VMEM((3, 8, 128), jnp.float32),
            pltpu.SemaphoreType.DMA((N_DEV - 1,)),
            pltpu.SemaphoreType.DMA((N_DEV - 1,)),
            pltpu.SemaphoreType.DMA((N_DEV - 1,)),
            pltpu.SemaphoreType.DMA((N_DEV - 1,)),
            pltpu.SemaphoreType.DMA((2,)),
            pltpu.SemaphoreType.DMA((2,)),
        ],
        compiler_params=pltpu.CompilerParams(
            collective_id=0, vmem_limit_bytes=100 * 1024 * 1024),
    )(x, w_mat)


# device time: 100236 ns/iter; 1.8695x vs baseline; 1.0893x over previous
import jax
import jax.numpy as jnp
from jax import lax
from jax.experimental import pallas as pl
from jax.experimental.pallas import tpu as pltpu

N_DEV = 4
N_Q = 4


def kernel(x, w_mat):
    m_total, k_per = x.shape
    _, n = w_mat.shape
    m_per = m_total // N_DEV
    qw = n // N_Q
    QORDER = (0, 2, 1, 3)

    def body(x_ref, w_ref, out_ref, sbuf, rbuf, abuf,
             send_sems, recv_sems, a_send, a_recv):
        my = lax.axis_index("i")
        left = lax.rem(my + N_DEV - 1, N_DEV)
        right = lax.rem(my + 1, N_DEV)

        barrier = pltpu.get_barrier_semaphore()
        for nbr in (left, right):
            pl.semaphore_signal(barrier, inc=1, device_id=(nbr,),
                                device_id_type=pl.DeviceIdType.MESH)
        pl.semaphore_wait(barrier, 2)

        wq = [w_ref[:, j * qw:(j + 1) * qw].astype(jnp.bfloat16)
              for j in range(N_Q)]

        def xchunk(c):
            return x_ref[pl.ds(c * m_per, m_per), :].astype(jnp.bfloat16)

        def part(xc, j):
            return jnp.dot(
                xc, wq[j], preferred_element_type=jnp.float32
            ).astype(jnp.bfloat16)

        def chunk_at(j, s):
            if j < 2:
                return lax.rem(my + 2 * N_DEV - 1 - s, N_DEV)
            return lax.rem(my + 1 + s, N_DEV)

        def mk(j, s):
            tgt = right if j < 2 else left
            return pltpu.make_async_remote_copy(
                src_ref=sbuf.at[j, s % 2], dst_ref=rbuf.at[j, s],
                send_sem=send_sems.at[j, s], recv_sem=recv_sems.at[j, s],
                device_id=(tgt,), device_id_type=pl.DeviceIdType.MESH)

        descs = {}
        xc_cw = xchunk(lax.rem(my + N_DEV - 1, N_DEV))
        xc_ccw = xchunk(lax.rem(my + 1, N_DEV))
        for j in QORDER:
            sbuf[j, 0] = part(xc_cw if j < 2 else xc_ccw, j)
            descs[(j, 0)] = mk(j, 0)
            descs[(j, 0)].start()

        for s in range(N_DEV - 1):
            if s < N_DEV - 2:
                nxt = [chunk_at(j, s + 1) for j in range(N_Q)]
            else:
                nxt = [my] * N_Q
            xc0 = xchunk(nxt[0])
            xc2 = xc0 if s == 0 or s == N_DEV - 2 else xchunk(nxt[2])
            p = [part(xc0 if j < 2 else xc2, j) for j in range(N_Q)]

            if s < N_DEV - 2:
                for j in QORDER:
                    d = descs[(j, s)]
                    if s >= 1:
                        descs[(j, s - 1)].wait_send()
                    d.wait_recv()
                    sbuf[j, (s + 1) % 2] = (
                        rbuf[j, s].astype(jnp.float32)
                        + p[j].astype(jnp.float32)).astype(jnp.bfloat16)
                    descs[(j, s + 1)] = mk(j, s + 1)
                    descs[(j, s + 1)].start()
            else:
                amax = None
                for j in QORDER:
                    descs[(j, s)].wait_recv()
                    y = jnp.maximum(
                        rbuf[j, s].astype(jnp.float32)
                        + p[j].astype(jnp.float32), 0.0)
                    m = jnp.max(y)
                    amax = m if amax is None else jnp.maximum(amax, m)
                    out_ref[:, j * qw:(j + 1) * qw] = y
        for j in range(N_Q):
            descs[(j, N_DEV - 3)].wait_send()
            descs[(j, N_DEV - 2)].wait_send()

        abuf[0] = jnp.full((8, 128), amax, jnp.float32)
        for t, partner in enumerate((my ^ 1, 3 - my)):
            rdma = pltpu.make_async_remote_copy(
                src_ref=abuf.at[0], dst_ref=abuf.at[t + 1],
                send_sem=a_send.at[t], recv_sem=a_recv.at[t],
                device_id=(partner,), device_id_type=pl.DeviceIdType.MESH)
            rdma.start()
            rdma.wait()
            abuf[0] = jnp.full(
                (8, 128),
                jnp.maximum(jnp.max(abuf[0]), jnp.max(abuf[t + 1])),
                jnp.float32)

        scale = jnp.max(abuf[0]) / 448.0
        inv = 1.0 / scale
        for j in range(N_Q):
            blk = out_ref[:, j * qw:(j + 1) * qw]
            qv = (blk * inv).astype(jnp.float8_e4m3fn)
            out_ref[:, j * qw:(j + 1) * qw] = qv.astype(jnp.float32) * scale

    return pl.pallas_call(
        body,
        out_shape=jax.ShapeDtypeStruct((m_per, n), jnp.float32),
        in_specs=[pl.BlockSpec(memory_space=pltpu.VMEM),
                  pl.BlockSpec(memory_space=pltpu.VMEM)],
        out_specs=pl.BlockSpec(memory_space=pltpu.VMEM),
        scratch_shapes=[
            pltpu.VMEM((N_Q, 2, m_per, qw), jnp.bfloat16),
            pltpu.VMEM((N_Q, N_DEV - 1, m_per, qw), jnp.bfloat16),
            pltpu.VMEM((3, 8, 128), jnp.float32),
            pltpu.SemaphoreType.DMA((N_Q, N_DEV - 1)),
            pltpu.SemaphoreType.DMA((N_Q, N_DEV - 1)),
            pltpu.SemaphoreType.DMA((2,)),
            pltpu.SemaphoreType.DMA((2,)),
        ],
        compiler_params=pltpu.CompilerParams(
            collective_id=0, vmem_limit_bytes=100 * 1024 * 1024),
    )(x, w_mat)
